# baseline (speedup 1.0000x reference)
import functools

import jax
import jax.numpy as jnp
from jax import lax
from jax.experimental import pallas as pl
from jax.experimental.pallas import tpu as pltpu
from jax.experimental.pallas import tpu_sc as plsc

N = 10000
NPAD = 10240
E = 320000
D_IN = 128
D_HID = 128
D_OUT = 16
LPA_ITER = 3

NC = 2
NS = 16
L = 16
ROWS_T = NPAD // NS
ROWS_W = NPAD // (NC * NS)

SUB = 100
HROWS = 100
EROWS = E // SUB
NB = 5
NBY = 2
D_Y = D_HID // NC

_i32 = jnp.int32
_f32 = jnp.float32


def _mm_body(x_ref, w_ref, o_ref):
    o_ref[...] = jnp.dot(x_ref[...], w_ref[...], preferred_element_type=_f32)


def _tc_matmul(x, w, bn=1024):
    m, k = x.shape
    _, n = w.shape
    return pl.pallas_call(
        _mm_body,
        grid=(m // bn,),
        in_specs=[
            pl.BlockSpec((bn, k), lambda i: (i, 0)),
            pl.BlockSpec((k, n), lambda i: (0, 0)),
        ],
        out_specs=pl.BlockSpec((bn, n), lambda i: (i, 0)),
        out_shape=jax.ShapeDtypeStruct((m, n), _f32),
    )(x, w)


def _hid_body(pa_ref, pb_ref, di_ref, b0_ref, w1_ref, u_ref):
    h = jnp.concatenate([pa_ref[...], pb_ref[...]], axis=1)
    h = di_ref[...] * h + b0_ref[...]
    h = jnp.maximum(h, 0.0)
    u_ref[...] = jnp.dot(h, w1_ref[...], preferred_element_type=_f32)


def _tc_hidden(pa, pb, di, b0, w1, bn=1024):
    return pl.pallas_call(
        _hid_body,
        grid=(NPAD // bn,),
        in_specs=[
            pl.BlockSpec((bn, D_Y), lambda i: (i, 0)),
            pl.BlockSpec((bn, D_Y), lambda i: (i, 0)),
            pl.BlockSpec((bn, 1), lambda i: (i, 0)),
            pl.BlockSpec((1, D_HID), lambda i: (0, 0)),
            pl.BlockSpec((D_HID, D_OUT), lambda i: (0, 0)),
        ],
        out_specs=pl.BlockSpec((bn, D_OUT), lambda i: (i, 0)),
        out_shape=jax.ShapeDtypeStruct((NPAD, D_OUT), _f32),
    )(pa, pb, di, b0, w1)


def _splat2(ref2d, s, j):
    zi = jnp.full((L,), 0, _i32)
    return plsc.load_gather(ref2d, [zi + s, zi + j])


def _sigmoid(v):
    return 1.0 / (1.0 + jnp.exp(-v))


@functools.cache
def _mesh():
    return plsc.VectorSubcoreMesh(core_axis_name="c", subcore_axis_name="s",
                                  num_cores=NC, num_subcores=NS)


_sc_params = pltpu.CompilerParams(needs_layout_passes=False,
                                  use_tc_tiling_on_sc=False)


@functools.cache
def _build_sc_main():
  @functools.partial(
    pl.kernel,
    out_type=(
        jax.ShapeDtypeStruct((NC * NPAD, D_Y), _f32),
        jax.ShapeDtypeStruct((NPAD, D_OUT), _f32),
        jax.ShapeDtypeStruct((NPAD,), _f32),
    ),
    scratch_types=(
        pltpu.VMEM_SHARED((NPAD, D_Y), _f32),
        pltpu.VMEM_SHARED((NPAD, D_OUT), _f32),
        pltpu.VMEM_SHARED((NPAD, D_OUT), _f32),
        pltpu.VMEM_SHARED((NPAD,), _f32),
        pltpu.VMEM((HROWS, SUB), _i32),
        pltpu.VMEM((HROWS, SUB), _i32),
        pltpu.VMEM((HROWS, SUB), _f32),
        pltpu.VMEM((NB, SUB), _i32),
        pltpu.VMEM((NB, SUB, D_OUT), _f32),
        pltpu.VMEM((NB, SUB, D_OUT), _f32),
        pltpu.VMEM((NBY, SUB, D_Y), _f32),
        pltpu.VMEM((ROWS_T,), _f32),
        pltpu.VMEM((ROWS_T // 2, D_OUT), _f32),
        pltpu.SemaphoreType.DMA,
        pltpu.SemaphoreType.DMA,
        pltpu.SemaphoreType.DMA,
        pltpu.SemaphoreType.DMA,
    ),
    mesh=_mesh(),
    compiler_params=_sc_params,
  )
  def _sc_main(row2_hbm, col2_hbm, attr2_hbm, aexp_hbm, z0f_hbm, lab0_hbm,
               y1p_hbm, labout_hbm, deginv_hbm,
               accy, acc16, labcur, degacc, rowvb, colvb, attrvb, colv2b,
               lbuf, abuf, ybuf, dinv, wb, esem, gsem, ssem, dsem):
    cid = lax.axis_index("c")
    sid = lax.axis_index("s")
    rbase = sid * ROWS_T
    z16 = jnp.zeros((L,), _f32)

    def _load_rows(base):
        d1 = pltpu.async_copy(row2_hbm.at[pl.ds(base, HROWS)], rowvb, esem)
        d2 = pltpu.async_copy(col2_hbm.at[pl.ds(base, HROWS)], colvb, esem)
        d3 = pltpu.async_copy(attr2_hbm.at[pl.ds(base, HROWS)], attrvb, esem)
        d1.wait()
        d2.wait()
        d3.wait()

    def _zero_acc16_slice():
        @pl.loop(0, SUB)
        def _(r):
            lbuf[0, r, :] = z16

        for off in (0, 100, 200, 300, 400, 500, 540):
            pltpu.sync_copy(lbuf.at[0], acc16.at[pl.ds(rbase + off, SUB)])

    def _lab_sweep(erow_base, with_deg):
        @pl.loop(0, HROWS // NB)
        def _(k):
            gds, ads, sds, dds = [], [], [], []
            for i in range(NB):
                s = k * NB + i
                gds.append(pltpu.async_copy(
                    labcur.at[colvb.at[s]], lbuf.at[i], gsem))
                ads.append(pltpu.async_copy(
                    aexp_hbm.at[erow_base + s], abuf.at[i], esem))
                if with_deg:
                    dds.append(pltpu.async_copy(
                        attrvb.at[s], degacc.at[rowvb.at[s]], dsem, add=True))
            for i in range(NB):
                s = k * NB + i
                gds[i].wait()
                ads[i].wait()

                @pl.loop(0, SUB, unroll=8)
                def _(j):
                    lbuf[i, j, :] = lbuf[i, j, :] * abuf[i, j, :]

                sds.append(pltpu.async_copy(lbuf.at[i], acc16.at[rowvb.at[s]],
                                            ssem, add=True))
            for d in sds:
                d.wait()
            for d in dds:
                d.wait()

    @pl.loop(0, ROWS_T // L)
    def _(m):
        dinv[pl.ds(m * L, L)] = z16

    @pl.loop(0, SUB)
    def _(r):
        for d in range(D_Y // L):
            ybuf[0, r, pl.ds(d * L, L)] = z16

    pltpu.sync_copy(dinv, degacc.at[pl.ds(rbase, ROWS_T)])
    _zero_acc16_slice()
    for off in (0, 100, 200, 300, 400, 500, 540):
        pltpu.sync_copy(ybuf.at[0], accy.at[pl.ds(rbase + off, SUB)])

    for p in range(2):
        pb = p * (ROWS_T // 2)
        pltpu.sync_copy(lab0_hbm.at[pl.ds(rbase + pb, ROWS_T // 2)], wb)
        pltpu.sync_copy(wb, labcur.at[pl.ds(rbase + pb, ROWS_T // 2)])
    plsc.subcore_barrier()

    _load_rows(sid * 2 * HROWS)
    _lab_sweep(sid * 2 * HROWS, True)
    _load_rows(sid * 2 * HROWS + HROWS)
    _lab_sweep(sid * 2 * HROWS + HROWS, True)
    plsc.subcore_barrier()

    pltpu.sync_copy(degacc.at[pl.ds(rbase, ROWS_T)], dinv)

    @pl.loop(0, ROWS_T // L)
    def _(m):
        s = pl.ds(m * L, L)
        d = dinv[s]
        dinv[s] = jnp.where(d > 0.0, 1.0 / d, 0.0)

    @pl.when(cid == 0)
    def _():
        pltpu.sync_copy(dinv, deginv_hbm.at[pl.ds(rbase, ROWS_T)])

    for it in range(LPA_ITER):
        if it > 0:
            for h in range(2):
                _load_rows(sid * 2 * HROWS + h * HROWS)
                _lab_sweep(sid * 2 * HROWS + h * HROWS, False)
            plsc.subcore_barrier()

        for p in range(2):
            pb = p * (ROWS_T // 2)
            pltpu.sync_copy(acc16.at[pl.ds(rbase + pb, ROWS_T // 2)], wb)
            if it == LPA_ITER - 1:
                @pl.loop(0, ROWS_T // 2, unroll=4)
                def _(r):
                    di = plsc.load_gather(
                        dinv, [jnp.full((L,), pb, _i32) + r])
                    wb[r, :] = _sigmoid(wb[r, :] * di)

                @pl.when(cid == 0)
                def _():
                    pltpu.sync_copy(
                        wb, labout_hbm.at[pl.ds(rbase + pb, ROWS_T // 2)])
            else:
                @pl.loop(0, ROWS_T // 2, unroll=4)
                def _(r):
                    di = plsc.load_gather(
                        dinv, [jnp.full((L,), pb, _i32) + r])
                    wb[r, :] = wb[r, :] * di

                pltpu.sync_copy(
                    wb, labcur.at[pl.ds(rbase + pb, ROWS_T // 2)])
        if it < LPA_ITER - 1:
            _zero_acc16_slice()
        plsc.subcore_barrier()

    for h in range(2):
        _load_rows(sid * 2 * HROWS + h * HROWS)

        @pl.loop(0, HROWS // NBY)
        def _(k):
            gds, sds = [], []
            for i in range(NBY):
                s = k * NBY + i

                @pl.loop(0, 7)
                def _(m):
                    sl = pl.ds(jnp.minimum(m * L, SUB - L), L)
                    colv2b[i, sl] = colvb[s, sl] + cid * NPAD

                gds.append(pltpu.async_copy(
                    z0f_hbm.at[colv2b.at[i]], ybuf.at[i], gsem))
            for i in range(NBY):
                s = k * NBY + i
                gds[i].wait()

                @pl.loop(0, SUB, unroll=2)
                def _(j):
                    a = _splat2(attrvb, s, j)
                    for d in range(D_Y // L):
                        sl = pl.ds(d * L, L)
                        ybuf[i, j, sl] = ybuf[i, j, sl] * a

                sds.append(pltpu.async_copy(ybuf.at[i], accy.at[rowvb.at[s]],
                                            ssem, add=True))
            for d in sds:
                d.wait()

    plsc.subcore_barrier()
    pltpu.sync_copy(accy.at[pl.ds(rbase, ROWS_T)],
                    y1p_hbm.at[pl.ds(cid * NPAD + rbase, ROWS_T)])

  return _sc_main


@functools.cache
def _build_sc_out():
  @functools.partial(
    pl.kernel,
    out_type=jax.ShapeDtypeStruct((NPAD, D_OUT), _f32),
    scratch_types=(
        pltpu.VMEM_SHARED((NPAD, D_OUT), _f32),
        pltpu.VMEM_SHARED((NPAD, D_OUT), _f32),
        pltpu.VMEM((HROWS, SUB), _i32),
        pltpu.VMEM((HROWS, SUB), _i32),
        pltpu.VMEM((HROWS, SUB), _f32),
        pltpu.VMEM((NB, SUB, D_OUT), _f32),
        pltpu.VMEM((NB, SUB, D_OUT), _f32),
        pltpu.VMEM((160, D_OUT), _f32),
        pltpu.VMEM((ROWS_W,), _f32),
        pltpu.VMEM((ROWS_W, D_OUT), _f32),
        pltpu.VMEM((L,), _f32),
        pltpu.SemaphoreType.DMA,
        pltpu.SemaphoreType.DMA,
        pltpu.SemaphoreType.DMA,
    ),
    mesh=_mesh(),
    compiler_params=_sc_params,
  )
  def _sc_out(row2_hbm, col2_hbm, attr2_hbm, aexp_hbm, u_hbm, deginv_hbm,
              b1_hbm, out_hbm, acc16, ucur, rowvb, colvb, attrvb, lbuf, abuf,
              zlab, dinv, wb, b1v, esem, gsem, ssem):
    cid = lax.axis_index("c")
    sid = lax.axis_index("s")
    rbase = sid * ROWS_T
    wbase = cid * (NPAD // NC) + sid * ROWS_W
    z16 = jnp.zeros((L,), _f32)

    @pl.loop(0, 160)
    def _(r):
        zlab[r, :] = z16

    for i in range(4):
        pltpu.sync_copy(zlab, acc16.at[pl.ds(rbase + i * 160, 160)])
    for p in range(2):
        pb = p * ROWS_W
        pltpu.sync_copy(u_hbm.at[pl.ds(rbase + pb, ROWS_W)], wb)
        pltpu.sync_copy(wb, ucur.at[pl.ds(rbase + pb, ROWS_W)])
    pltpu.sync_copy(deginv_hbm.at[pl.ds(wbase, ROWS_W)], dinv)
    pltpu.sync_copy(b1_hbm, b1v)
    plsc.subcore_barrier()

    for h in range(2):
        d1 = pltpu.async_copy(
            row2_hbm.at[pl.ds(sid * 2 * HROWS + h * HROWS, HROWS)], rowvb, esem)
        d2 = pltpu.async_copy(
            col2_hbm.at[pl.ds(sid * 2 * HROWS + h * HROWS, HROWS)], colvb, esem)
        d3 = pltpu.async_copy(
            attr2_hbm.at[pl.ds(sid * 2 * HROWS + h * HROWS, HROWS)], attrvb, esem)
        d1.wait()
        d2.wait()
        d3.wait()

        @pl.loop(0, HROWS // NB)
        def _(k):
            gds, ads, sds = [], [], []
            for i in range(NB):
                s = k * NB + i
                gds.append(pltpu.async_copy(
                    ucur.at[colvb.at[s]], lbuf.at[i], gsem))
                ads.append(pltpu.async_copy(
                    aexp_hbm.at[sid * 2 * HROWS + h * HROWS + s],
                    abuf.at[i], esem))
            for i in range(NB):
                s = k * NB + i
                gds[i].wait()
                ads[i].wait()

                @pl.loop(0, SUB, unroll=8)
                def _(j):
                    lbuf[i, j, :] = lbuf[i, j, :] * abuf[i, j, :]

                sds.append(pltpu.async_copy(lbuf.at[i], acc16.at[rowvb.at[s]],
                                            ssem, add=True))
            for d in sds:
                d.wait()

    plsc.subcore_barrier()
    pltpu.sync_copy(acc16.at[pl.ds(wbase, ROWS_W)], wb)

    b1r = b1v[...]

    @pl.loop(0, ROWS_W, unroll=4)
    def _(r):
        di = plsc.load_gather(dinv, [jnp.full((L,), 0, _i32) + r])
        wb[r, :] = _sigmoid(wb[r, :] * di + b1r)

    pltpu.sync_copy(wb, out_hbm.at[pl.ds(wbase, ROWS_W)])

  return _sc_out


def kernel(x, soft_labels, edge_index, edge_attr, W0, b0, W1, b1):
    row2 = edge_index[0].reshape(EROWS, SUB)
    col2 = edge_index[1].reshape(EROWS, SUB)
    attr2 = edge_attr.reshape(EROWS, SUB)
    aexp = jnp.broadcast_to(edge_attr[:, None], (E, D_OUT)).reshape(
        EROWS, SUB, D_OUT)
    xp = jnp.pad(x, ((0, NPAD - N), (0, 0)))
    labp = jnp.pad(soft_labels, ((0, NPAD - N), (0, 0)))

    z0 = _tc_matmul(xp, W0)
    z0f = jnp.concatenate([z0[:, :D_Y], z0[:, D_Y:]], axis=0)
    y1p, labout, deginv = _build_sc_main()(row2, col2, attr2, aexp, z0f, labp)
    u = _tc_hidden(y1p[:NPAD], y1p[NPAD:], deginv.reshape(NPAD, 1),
                   b0.reshape(1, D_HID), W1)
    outp = _build_sc_out()(row2, col2, attr2, aexp, u, deginv, b1)
    return outp[:N], labout[:N]


# speedup vs baseline: 1.1541x; 1.1541x over previous
import functools

import jax
import jax.numpy as jnp
from jax import lax
from jax.experimental import pallas as pl
from jax.experimental.pallas import tpu as pltpu
from jax.experimental.pallas import tpu_sc as plsc

N = 10000
NPAD = 10240
E = 320000
D_IN = 128
D_HID = 128
D_OUT = 16
LPA_ITER = 3

NC = 2
NS = 16
L = 16
ROWS_T = NPAD // NS
ROWS_W = NPAD // (NC * NS)

SUB = 100
HROWS = 100
EROWS = E // SUB
NB = 10
NBY = 2
D_Y = D_HID // NC

_i32 = jnp.int32
_f32 = jnp.float32


def _mm_body(x_ref, w_ref, o_ref):
    o_ref[...] = jnp.dot(x_ref[...], w_ref[...], preferred_element_type=_f32)


def _tc_matmul(x, w, bn=1024):
    m, k = x.shape
    _, n = w.shape
    return pl.pallas_call(
        _mm_body,
        grid=(m // bn,),
        in_specs=[
            pl.BlockSpec((bn, k), lambda i: (i, 0)),
            pl.BlockSpec((k, n), lambda i: (0, 0)),
        ],
        out_specs=pl.BlockSpec((bn, n), lambda i: (i, 0)),
        out_shape=jax.ShapeDtypeStruct((m, n), _f32),
    )(x, w)


def _hid_body(pa_ref, pb_ref, di_ref, b0_ref, w1_ref, u_ref):
    h = jnp.concatenate([pa_ref[...], pb_ref[...]], axis=1)
    h = di_ref[...] * h + b0_ref[...]
    h = jnp.maximum(h, 0.0)
    u_ref[...] = jnp.dot(h, w1_ref[...], preferred_element_type=_f32)


def _tc_hidden(pa, pb, di, b0, w1, bn=1024):
    return pl.pallas_call(
        _hid_body,
        grid=(NPAD // bn,),
        in_specs=[
            pl.BlockSpec((bn, D_Y), lambda i: (i, 0)),
            pl.BlockSpec((bn, D_Y), lambda i: (i, 0)),
            pl.BlockSpec((bn, 1), lambda i: (i, 0)),
            pl.BlockSpec((1, D_HID), lambda i: (0, 0)),
            pl.BlockSpec((D_HID, D_OUT), lambda i: (0, 0)),
        ],
        out_specs=pl.BlockSpec((bn, D_OUT), lambda i: (i, 0)),
        out_shape=jax.ShapeDtypeStruct((NPAD, D_OUT), _f32),
    )(pa, pb, di, b0, w1)


def _splat2(ref2d, s, j):
    zi = jnp.full((L,), 0, _i32)
    return plsc.load_gather(ref2d, [zi + s, zi + j])


def _sigmoid(v):
    return 1.0 / (1.0 + jnp.exp(-v))


@functools.cache
def _mesh():
    return plsc.VectorSubcoreMesh(core_axis_name="c", subcore_axis_name="s",
                                  num_cores=NC, num_subcores=NS)


_sc_params = pltpu.CompilerParams(needs_layout_passes=False,
                                  use_tc_tiling_on_sc=False)


@functools.cache
def _build_sc_main():
  @functools.partial(
    pl.kernel,
    out_type=(
        jax.ShapeDtypeStruct((NC * NPAD, D_Y), _f32),
        jax.ShapeDtypeStruct((NPAD, D_OUT), _f32),
        jax.ShapeDtypeStruct((NPAD,), _f32),
    ),
    scratch_types=(
        pltpu.VMEM_SHARED((NPAD, D_Y), _f32),
        pltpu.VMEM_SHARED((NPAD, D_OUT), _f32),
        pltpu.VMEM_SHARED((NPAD, D_OUT), _f32),
        pltpu.VMEM_SHARED((NPAD,), _f32),
        pltpu.VMEM((HROWS, SUB), _i32),
        pltpu.VMEM((HROWS, SUB), _i32),
        pltpu.VMEM((HROWS, SUB), _f32),
        pltpu.VMEM((NB, SUB), _i32),
        pltpu.VMEM((NB, SUB, D_OUT), _f32),
        pltpu.VMEM((NBY, SUB, D_Y), _f32),
        pltpu.VMEM((ROWS_T,), _f32),
        pltpu.VMEM((ROWS_T // 2, D_OUT), _f32),
        pltpu.SemaphoreType.DMA,
        pltpu.SemaphoreType.DMA,
        pltpu.SemaphoreType.DMA,
        pltpu.SemaphoreType.DMA,
    ),
    mesh=_mesh(),
    compiler_params=_sc_params,
  )
  def _sc_main(row2_hbm, col2_hbm, attr2_hbm, z0f_hbm, lab0_hbm,
               y1p_hbm, labout_hbm, deginv_hbm,
               accy, acc16, labcur, degacc, rowvb, colvb, attrvb, colv2b,
               lbuf, ybuf, dinv, wb, esem, gsem, ssem, dsem):
    cid = lax.axis_index("c")
    sid = lax.axis_index("s")
    rbase = sid * ROWS_T
    z16 = jnp.zeros((L,), _f32)

    def _load_rows(base):
        d1 = pltpu.async_copy(row2_hbm.at[pl.ds(base, HROWS)], rowvb, esem)
        d2 = pltpu.async_copy(col2_hbm.at[pl.ds(base, HROWS)], colvb, esem)
        d3 = pltpu.async_copy(attr2_hbm.at[pl.ds(base, HROWS)], attrvb, esem)
        d1.wait()
        d2.wait()
        d3.wait()

    def _zero_acc16_slice():
        @pl.loop(0, SUB)
        def _(r):
            lbuf[0, r, :] = z16

        for off in (0, 100, 200, 300, 400, 500, 540):
            pltpu.sync_copy(lbuf.at[0], acc16.at[pl.ds(rbase + off, SUB)])

    def _lab_sweep(with_deg):
        @pl.loop(0, HROWS // NB)
        def _(k):
            @pl.when(k > 0)
            def _():
                for i in range(NB):
                    pltpu.make_async_copy(
                        lab0_hbm.at[pl.ds(0, SUB)], lbuf.at[i], ssem).wait()
            gds, dds = [], []
            for i in range(NB):
                s = k * NB + i
                gds.append(pltpu.async_copy(
                    labcur.at[colvb.at[s]], lbuf.at[i], gsem))
                if with_deg:
                    dds.append(pltpu.async_copy(
                        attrvb.at[s], degacc.at[rowvb.at[s]], dsem, add=True))
            for i in range(NB):
                s = k * NB + i
                gds[i].wait()

                @pl.loop(0, SUB, unroll=4)
                def _(j):
                    a = _splat2(attrvb, s, j)
                    lbuf[i, j, :] = lbuf[i, j, :] * a

                pltpu.async_copy(lbuf.at[i], acc16.at[rowvb.at[s]],
                                 ssem, add=True)
            for d in dds:
                d.wait()

    def _lab_drain():
        for i in range(NB):
            pltpu.make_async_copy(
                lab0_hbm.at[pl.ds(0, SUB)], lbuf.at[i], ssem).wait()

    @pl.loop(0, ROWS_T // L)
    def _(m):
        dinv[pl.ds(m * L, L)] = z16

    @pl.loop(0, SUB)
    def _(r):
        for d in range(D_Y // L):
            ybuf[0, r, pl.ds(d * L, L)] = z16

    pltpu.sync_copy(dinv, degacc.at[pl.ds(rbase, ROWS_T)])
    _zero_acc16_slice()
    for off in (0, 100, 200, 300, 400, 500, 540):
        pltpu.sync_copy(ybuf.at[0], accy.at[pl.ds(rbase + off, SUB)])

    for p in range(2):
        pb = p * (ROWS_T // 2)
        pltpu.sync_copy(lab0_hbm.at[pl.ds(rbase + pb, ROWS_T // 2)], wb)
        pltpu.sync_copy(wb, labcur.at[pl.ds(rbase + pb, ROWS_T // 2)])
    plsc.subcore_barrier()

    _load_rows(sid * 2 * HROWS)
    _lab_sweep(True)
    _lab_drain()
    _load_rows(sid * 2 * HROWS + HROWS)
    _lab_sweep(True)
    _lab_drain()
    plsc.subcore_barrier()

    pltpu.sync_copy(degacc.at[pl.ds(rbase, ROWS_T)], dinv)

    @pl.loop(0, ROWS_T // L)
    def _(m):
        s = pl.ds(m * L, L)
        d = dinv[s]
        dinv[s] = jnp.where(d > 0.0, 1.0 / d, 0.0)

    @pl.when(cid == 0)
    def _():
        pltpu.sync_copy(dinv, deginv_hbm.at[pl.ds(rbase, ROWS_T)])

    for it in range(LPA_ITER):
        if it > 0:
            for h in range(2):
                _load_rows(sid * 2 * HROWS + h * HROWS)
                _lab_sweep(False)
                _lab_drain()
            plsc.subcore_barrier()

        for p in range(2):
            pb = p * (ROWS_T // 2)
            pltpu.sync_copy(acc16.at[pl.ds(rbase + pb, ROWS_T // 2)], wb)
            if it == LPA_ITER - 1:
                @pl.loop(0, ROWS_T // 2, unroll=4)
                def _(r):
                    di = plsc.load_gather(
                        dinv, [jnp.full((L,), pb, _i32) + r])
                    wb[r, :] = _sigmoid(wb[r, :] * di)

                @pl.when(cid == 0)
                def _():
                    pltpu.sync_copy(
                        wb, labout_hbm.at[pl.ds(rbase + pb, ROWS_T // 2)])
            else:
                @pl.loop(0, ROWS_T // 2, unroll=4)
                def _(r):
                    di = plsc.load_gather(
                        dinv, [jnp.full((L,), pb, _i32) + r])
                    wb[r, :] = wb[r, :] * di

                pltpu.sync_copy(
                    wb, labcur.at[pl.ds(rbase + pb, ROWS_T // 2)])
        if it < LPA_ITER - 1:
            _zero_acc16_slice()
        plsc.subcore_barrier()

    for h in range(2):
        _load_rows(sid * 2 * HROWS + h * HROWS)

        @pl.loop(0, HROWS // NBY)
        def _(k):
            @pl.when(k > 0)
            def _():
                for i in range(NBY):
                    pltpu.make_async_copy(
                        z0f_hbm.at[pl.ds(0, SUB)], ybuf.at[i], ssem).wait()
            gds = []
            for i in range(NBY):
                s = k * NBY + i

                @pl.loop(0, 7)
                def _(m):
                    sl = pl.ds(jnp.minimum(m * L, SUB - L), L)
                    colv2b[i, sl] = colvb[s, sl] + cid * NPAD

                gds.append(pltpu.async_copy(
                    z0f_hbm.at[colv2b.at[i]], ybuf.at[i], gsem))
            for i in range(NBY):
                s = k * NBY + i
                gds[i].wait()

                @pl.loop(0, SUB, unroll=2)
                def _(j):
                    a = _splat2(attrvb, s, j)
                    for d in range(D_Y // L):
                        sl = pl.ds(d * L, L)
                        ybuf[i, j, sl] = ybuf[i, j, sl] * a

                pltpu.async_copy(ybuf.at[i], accy.at[rowvb.at[s]],
                                 ssem, add=True)

        for i in range(NBY):
            pltpu.make_async_copy(
                z0f_hbm.at[pl.ds(0, SUB)], ybuf.at[i], ssem).wait()

    plsc.subcore_barrier()
    pltpu.sync_copy(accy.at[pl.ds(rbase, ROWS_T)],
                    y1p_hbm.at[pl.ds(cid * NPAD + rbase, ROWS_T)])

  return _sc_main


@functools.cache
def _build_sc_out():
  @functools.partial(
    pl.kernel,
    out_type=jax.ShapeDtypeStruct((NPAD, D_OUT), _f32),
    scratch_types=(
        pltpu.VMEM_SHARED((NPAD, D_OUT), _f32),
        pltpu.VMEM_SHARED((NPAD, D_OUT), _f32),
        pltpu.VMEM((HROWS, SUB), _i32),
        pltpu.VMEM((HROWS, SUB), _i32),
        pltpu.VMEM((HROWS, SUB), _f32),
        pltpu.VMEM((NB, SUB, D_OUT), _f32),
        pltpu.VMEM((160, D_OUT), _f32),
        pltpu.VMEM((ROWS_W,), _f32),
        pltpu.VMEM((ROWS_W, D_OUT), _f32),
        pltpu.VMEM((L,), _f32),
        pltpu.SemaphoreType.DMA,
        pltpu.SemaphoreType.DMA,
        pltpu.SemaphoreType.DMA,
    ),
    mesh=_mesh(),
    compiler_params=_sc_params,
  )
  def _sc_out(row2_hbm, col2_hbm, attr2_hbm, u_hbm, deginv_hbm,
              b1_hbm, out_hbm, acc16, ucur, rowvb, colvb, attrvb, lbuf,
              zlab, dinv, wb, b1v, esem, gsem, ssem):
    cid = lax.axis_index("c")
    sid = lax.axis_index("s")
    rbase = sid * ROWS_T
    wbase = cid * (NPAD // NC) + sid * ROWS_W
    z16 = jnp.zeros((L,), _f32)

    @pl.loop(0, 160)
    def _(r):
        zlab[r, :] = z16

    for i in range(4):
        pltpu.sync_copy(zlab, acc16.at[pl.ds(rbase + i * 160, 160)])
    for p in range(2):
        pb = p * ROWS_W
        pltpu.sync_copy(u_hbm.at[pl.ds(rbase + pb, ROWS_W)], wb)
        pltpu.sync_copy(wb, ucur.at[pl.ds(rbase + pb, ROWS_W)])
    pltpu.sync_copy(deginv_hbm.at[pl.ds(wbase, ROWS_W)], dinv)
    pltpu.sync_copy(b1_hbm, b1v)
    plsc.subcore_barrier()

    for h in range(2):
        d1 = pltpu.async_copy(
            row2_hbm.at[pl.ds(sid * 2 * HROWS + h * HROWS, HROWS)], rowvb, esem)
        d2 = pltpu.async_copy(
            col2_hbm.at[pl.ds(sid * 2 * HROWS + h * HROWS, HROWS)], colvb, esem)
        d3 = pltpu.async_copy(
            attr2_hbm.at[pl.ds(sid * 2 * HROWS + h * HROWS, HROWS)], attrvb, esem)
        d1.wait()
        d2.wait()
        d3.wait()

        @pl.loop(0, HROWS // NB)
        def _(k):
            @pl.when(k > 0)
            def _():
                for i in range(NB):
                    pltpu.make_async_copy(
                        u_hbm.at[pl.ds(0, SUB)], lbuf.at[i], ssem).wait()
            gds = []
            for i in range(NB):
                s = k * NB + i
                gds.append(pltpu.async_copy(
                    ucur.at[colvb.at[s]], lbuf.at[i], gsem))
            for i in range(NB):
                s = k * NB + i
                gds[i].wait()

                @pl.loop(0, SUB, unroll=4)
                def _(j):
                    a = _splat2(attrvb, s, j)
                    lbuf[i, j, :] = lbuf[i, j, :] * a

                pltpu.async_copy(lbuf.at[i], acc16.at[rowvb.at[s]],
                                 ssem, add=True)

        for i in range(NB):
            pltpu.make_async_copy(
                u_hbm.at[pl.ds(0, SUB)], lbuf.at[i], ssem).wait()

    plsc.subcore_barrier()
    pltpu.sync_copy(acc16.at[pl.ds(wbase, ROWS_W)], wb)

    b1r = b1v[...]

    @pl.loop(0, ROWS_W, unroll=4)
    def _(r):
        di = plsc.load_gather(dinv, [jnp.full((L,), 0, _i32) + r])
        wb[r, :] = _sigmoid(wb[r, :] * di + b1r)

    pltpu.sync_copy(wb, out_hbm.at[pl.ds(wbase, ROWS_W)])

  return _sc_out


def kernel(x, soft_labels, edge_index, edge_attr, W0, b0, W1, b1):
    row2 = edge_index[0].reshape(EROWS, SUB)
    col2 = edge_index[1].reshape(EROWS, SUB)
    attr2 = edge_attr.reshape(EROWS, SUB)
    xp = jnp.pad(x, ((0, NPAD - N), (0, 0)))
    labp = jnp.pad(soft_labels, ((0, NPAD - N), (0, 0)))

    z0 = _tc_matmul(xp, W0)
    z0f = jnp.concatenate([z0[:, :D_Y], z0[:, D_Y:]], axis=0)
    y1p, labout, deginv = _build_sc_main()(row2, col2, attr2, z0f, labp)
    u = _tc_hidden(y1p[:NPAD], y1p[NPAD:], deginv.reshape(NPAD, 1),
                   b0.reshape(1, D_HID), W1)
    outp = _build_sc_out()(row2, col2, attr2, u, deginv, b1)
    return outp[:N], labout[:N]


# speedup vs baseline: 1.2388x; 1.0733x over previous
import functools

import jax
import jax.numpy as jnp
from jax import lax
from jax.experimental import pallas as pl
from jax.experimental.pallas import tpu as pltpu
from jax.experimental.pallas import tpu_sc as plsc

N = 10000
NPAD = 10240
E = 320000
D_IN = 128
D_HID = 128
D_OUT = 16
LPA_ITER = 3

NC = 2
NS = 16
L = 16
ROWS_T = NPAD // NS
ROWS_W = NPAD // (NC * NS)

SUB = 100
HROWS = 50
EROWS = E // SUB
NQ = 4
NB = 10
NBY = 5
D_Y = D_HID // NC

_i32 = jnp.int32
_f32 = jnp.float32


def _mm_body(x_ref, w_ref, o_ref):
    o_ref[...] = jnp.dot(x_ref[...], w_ref[...], preferred_element_type=_f32)


def _tc_matmul(x, w, bn=1024):
    m, k = x.shape
    _, n = w.shape
    return pl.pallas_call(
        _mm_body,
        grid=(m // bn,),
        in_specs=[
            pl.BlockSpec((bn, k), lambda i: (i, 0)),
            pl.BlockSpec((k, n), lambda i: (0, 0)),
        ],
        out_specs=pl.BlockSpec((bn, n), lambda i: (i, 0)),
        out_shape=jax.ShapeDtypeStruct((m, n), _f32),
    )(x, w)


def _hid_body(pa_ref, pb_ref, di_ref, b0_ref, w1_ref, u_ref):
    h = jnp.concatenate([pa_ref[...], pb_ref[...]], axis=1)
    h = di_ref[...] * h + b0_ref[...]
    h = jnp.maximum(h, 0.0)
    u_ref[...] = jnp.dot(h, w1_ref[...], preferred_element_type=_f32)


def _tc_hidden(pa, pb, di, b0, w1, bn=1024):
    return pl.pallas_call(
        _hid_body,
        grid=(NPAD // bn,),
        in_specs=[
            pl.BlockSpec((bn, D_Y), lambda i: (i, 0)),
            pl.BlockSpec((bn, D_Y), lambda i: (i, 0)),
            pl.BlockSpec((bn, 1), lambda i: (i, 0)),
            pl.BlockSpec((1, D_HID), lambda i: (0, 0)),
            pl.BlockSpec((D_HID, D_OUT), lambda i: (0, 0)),
        ],
        out_specs=pl.BlockSpec((bn, D_OUT), lambda i: (i, 0)),
        out_shape=jax.ShapeDtypeStruct((NPAD, D_OUT), _f32),
    )(pa, pb, di, b0, w1)


def _splat2(ref2d, s, j):
    zi = jnp.full((L,), 0, _i32)
    return plsc.load_gather(ref2d, [zi + s, zi + j])


def _sigmoid(v):
    return 1.0 / (1.0 + jnp.exp(-v))


@functools.cache
def _mesh():
    return plsc.VectorSubcoreMesh(core_axis_name="c", subcore_axis_name="s",
                                  num_cores=NC, num_subcores=NS)


_sc_params = pltpu.CompilerParams(needs_layout_passes=False,
                                  use_tc_tiling_on_sc=False)


@functools.cache
def _build_sc_main():
  @functools.partial(
    pl.kernel,
    out_type=(
        jax.ShapeDtypeStruct((NC * NPAD, D_Y), _f32),
        jax.ShapeDtypeStruct((NPAD, D_OUT), _f32),
        jax.ShapeDtypeStruct((NPAD,), _f32),
    ),
    scratch_types=(
        pltpu.VMEM_SHARED((NPAD, D_Y), _f32),
        pltpu.VMEM_SHARED((NPAD, D_OUT), _f32),
        pltpu.VMEM_SHARED((NPAD, D_OUT), _f32),
        pltpu.VMEM_SHARED((NPAD,), _f32),
        pltpu.VMEM((HROWS, SUB), _i32),
        pltpu.VMEM((HROWS, SUB), _i32),
        pltpu.VMEM((HROWS, SUB), _f32),
        pltpu.VMEM((NB, SUB), _i32),
        pltpu.VMEM((NB, SUB, D_OUT), _f32),
        pltpu.VMEM((NBY, SUB, D_Y), _f32),
        pltpu.VMEM((ROWS_T,), _f32),
        pltpu.VMEM((ROWS_T // 4, D_OUT), _f32),
        pltpu.SemaphoreType.DMA,
        pltpu.SemaphoreType.DMA,
        pltpu.SemaphoreType.DMA,
        pltpu.SemaphoreType.DMA,
    ),
    mesh=_mesh(),
    compiler_params=_sc_params,
  )
  def _sc_main(row2_hbm, col2_hbm, attr2_hbm, z0f_hbm, lab0_hbm,
               y1p_hbm, labout_hbm, deginv_hbm,
               accy, acc16, labcur, degacc, rowvb, colvb, attrvb, colv2b,
               lbuf, ybuf, dinv, wb, esem, gsem, ssem, dsem):
    cid = lax.axis_index("c")
    sid = lax.axis_index("s")
    rbase = sid * ROWS_T
    z16 = jnp.zeros((L,), _f32)

    def _load_rows(base):
        d1 = pltpu.async_copy(row2_hbm.at[pl.ds(base, HROWS)], rowvb, esem)
        d2 = pltpu.async_copy(col2_hbm.at[pl.ds(base, HROWS)], colvb, esem)
        d3 = pltpu.async_copy(attr2_hbm.at[pl.ds(base, HROWS)], attrvb, esem)
        d1.wait()
        d2.wait()
        d3.wait()

    def _zero_acc16_slice():
        @pl.loop(0, SUB)
        def _(r):
            lbuf[0, r, :] = z16

        for off in (0, 100, 200, 300, 400, 500, 540):
            pltpu.sync_copy(lbuf.at[0], acc16.at[pl.ds(rbase + off, SUB)])

    def _lab_sweep(with_deg):
        @pl.loop(0, HROWS // NB)
        def _(k):
            @pl.when(k > 0)
            def _():
                for i in range(NB):
                    pltpu.make_async_copy(
                        lab0_hbm.at[pl.ds(0, SUB)], lbuf.at[i], ssem).wait()
            gds, dds = [], []
            for i in range(NB):
                s = k * NB + i
                gds.append(pltpu.async_copy(
                    labcur.at[colvb.at[s]], lbuf.at[i], gsem))
                if with_deg:
                    dds.append(pltpu.async_copy(
                        attrvb.at[s], degacc.at[rowvb.at[s]], dsem, add=True))
            for i in range(NB):
                s = k * NB + i
                gds[i].wait()

                @pl.loop(0, SUB, unroll=4)
                def _(j):
                    a = _splat2(attrvb, s, j)
                    lbuf[i, j, :] = lbuf[i, j, :] * a

                pltpu.async_copy(lbuf.at[i], acc16.at[rowvb.at[s]],
                                 ssem, add=True)
            for d in dds:
                d.wait()

    def _lab_drain():
        for i in range(NB):
            pltpu.make_async_copy(
                lab0_hbm.at[pl.ds(0, SUB)], lbuf.at[i], ssem).wait()

    @pl.loop(0, ROWS_T // L)
    def _(m):
        dinv[pl.ds(m * L, L)] = z16

    @pl.loop(0, SUB)
    def _(r):
        for d in range(D_Y // L):
            ybuf[0, r, pl.ds(d * L, L)] = z16

    pltpu.sync_copy(dinv, degacc.at[pl.ds(rbase, ROWS_T)])
    _zero_acc16_slice()
    for off in (0, 100, 200, 300, 400, 500, 540):
        pltpu.sync_copy(ybuf.at[0], accy.at[pl.ds(rbase + off, SUB)])

    for p in range(4):
        pb = p * (ROWS_T // 4)
        pltpu.sync_copy(lab0_hbm.at[pl.ds(rbase + pb, ROWS_T // 4)], wb)
        pltpu.sync_copy(wb, labcur.at[pl.ds(rbase + pb, ROWS_T // 4)])
    plsc.subcore_barrier()

    @pl.loop(0, NQ)
    def _(h):
        _load_rows(sid * NQ * HROWS + h * HROWS)
        _lab_sweep(True)
        _lab_drain()

    plsc.subcore_barrier()

    pltpu.sync_copy(degacc.at[pl.ds(rbase, ROWS_T)], dinv)

    @pl.loop(0, ROWS_T // L)
    def _(m):
        s = pl.ds(m * L, L)
        d = dinv[s]
        dinv[s] = jnp.where(d > 0.0, 1.0 / d, 0.0)

    @pl.when(cid == 0)
    def _():
        pltpu.sync_copy(dinv, deginv_hbm.at[pl.ds(rbase, ROWS_T)])

    for it in range(LPA_ITER):
        if it > 0:
            @pl.loop(0, NQ)
            def _(h):
                _load_rows(sid * NQ * HROWS + h * HROWS)
                _lab_sweep(False)
                _lab_drain()

            plsc.subcore_barrier()

        for p in range(4):
            pb = p * (ROWS_T // 4)
            pltpu.sync_copy(acc16.at[pl.ds(rbase + pb, ROWS_T // 4)], wb)
            if it == LPA_ITER - 1:
                @pl.loop(0, ROWS_T // 4, unroll=4)
                def _(r):
                    di = plsc.load_gather(
                        dinv, [jnp.full((L,), pb, _i32) + r])
                    wb[r, :] = _sigmoid(wb[r, :] * di)

                @pl.when(cid == 0)
                def _():
                    pltpu.sync_copy(
                        wb, labout_hbm.at[pl.ds(rbase + pb, ROWS_T // 4)])
            else:
                @pl.loop(0, ROWS_T // 4, unroll=4)
                def _(r):
                    di = plsc.load_gather(
                        dinv, [jnp.full((L,), pb, _i32) + r])
                    wb[r, :] = wb[r, :] * di

                pltpu.sync_copy(
                    wb, labcur.at[pl.ds(rbase + pb, ROWS_T // 4)])
        if it < LPA_ITER - 1:
            _zero_acc16_slice()
        plsc.subcore_barrier()

    @pl.loop(0, NQ)
    def _(h):
        _load_rows(sid * NQ * HROWS + h * HROWS)

        @pl.loop(0, HROWS // NBY)
        def _(k):
            @pl.when(k > 0)
            def _():
                for i in range(NBY):
                    pltpu.make_async_copy(
                        z0f_hbm.at[pl.ds(0, SUB)], ybuf.at[i], ssem).wait()
            gds = []
            for i in range(NBY):
                s = k * NBY + i

                @pl.loop(0, 7)
                def _(m):
                    sl = pl.ds(jnp.minimum(m * L, SUB - L), L)
                    colv2b[i, sl] = colvb[s, sl] + cid * NPAD

                gds.append(pltpu.async_copy(
                    z0f_hbm.at[colv2b.at[i]], ybuf.at[i], gsem))
            for i in range(NBY):
                s = k * NBY + i
                gds[i].wait()

                @pl.loop(0, SUB, unroll=2)
                def _(j):
                    a = _splat2(attrvb, s, j)
                    for d in range(D_Y // L):
                        sl = pl.ds(d * L, L)
                        ybuf[i, j, sl] = ybuf[i, j, sl] * a

                pltpu.async_copy(ybuf.at[i], accy.at[rowvb.at[s]],
                                 ssem, add=True)

        for i in range(NBY):
            pltpu.make_async_copy(
                z0f_hbm.at[pl.ds(0, SUB)], ybuf.at[i], ssem).wait()

    plsc.subcore_barrier()
    pltpu.sync_copy(accy.at[pl.ds(rbase, ROWS_T)],
                    y1p_hbm.at[pl.ds(cid * NPAD + rbase, ROWS_T)])

  return _sc_main


@functools.cache
def _build_sc_out():
  @functools.partial(
    pl.kernel,
    out_type=jax.ShapeDtypeStruct((NPAD, D_OUT), _f32),
    scratch_types=(
        pltpu.VMEM_SHARED((NPAD, D_OUT), _f32),
        pltpu.VMEM_SHARED((NPAD, D_OUT), _f32),
        pltpu.VMEM((HROWS, SUB), _i32),
        pltpu.VMEM((HROWS, SUB), _i32),
        pltpu.VMEM((HROWS, SUB), _f32),
        pltpu.VMEM((NB, SUB, D_OUT), _f32),
        pltpu.VMEM((160, D_OUT), _f32),
        pltpu.VMEM((ROWS_W,), _f32),
        pltpu.VMEM((ROWS_W, D_OUT), _f32),
        pltpu.VMEM((L,), _f32),
        pltpu.SemaphoreType.DMA,
        pltpu.SemaphoreType.DMA,
        pltpu.SemaphoreType.DMA,
    ),
    mesh=_mesh(),
    compiler_params=_sc_params,
  )
  def _sc_out(row2_hbm, col2_hbm, attr2_hbm, u_hbm, deginv_hbm,
              b1_hbm, out_hbm, acc16, ucur, rowvb, colvb, attrvb, lbuf,
              zlab, dinv, wb, b1v, esem, gsem, ssem):
    cid = lax.axis_index("c")
    sid = lax.axis_index("s")
    rbase = sid * ROWS_T
    wbase = cid * (NPAD // NC) + sid * ROWS_W
    z16 = jnp.zeros((L,), _f32)

    @pl.loop(0, 160)
    def _(r):
        zlab[r, :] = z16

    for i in range(4):
        pltpu.sync_copy(zlab, acc16.at[pl.ds(rbase + i * 160, 160)])
    for p in range(2):
        pb = p * ROWS_W
        pltpu.sync_copy(u_hbm.at[pl.ds(rbase + pb, ROWS_W)], wb)
        pltpu.sync_copy(wb, ucur.at[pl.ds(rbase + pb, ROWS_W)])
    pltpu.sync_copy(deginv_hbm.at[pl.ds(wbase, ROWS_W)], dinv)
    pltpu.sync_copy(b1_hbm, b1v)
    plsc.subcore_barrier()

    for h in range(NQ):
        d1 = pltpu.async_copy(
            row2_hbm.at[pl.ds(sid * NQ * HROWS + h * HROWS, HROWS)], rowvb, esem)
        d2 = pltpu.async_copy(
            col2_hbm.at[pl.ds(sid * NQ * HROWS + h * HROWS, HROWS)], colvb, esem)
        d3 = pltpu.async_copy(
            attr2_hbm.at[pl.ds(sid * NQ * HROWS + h * HROWS, HROWS)], attrvb, esem)
        d1.wait()
        d2.wait()
        d3.wait()

        @pl.loop(0, HROWS // NB)
        def _(k):
            @pl.when(k > 0)
            def _():
                for i in range(NB):
                    pltpu.make_async_copy(
                        u_hbm.at[pl.ds(0, SUB)], lbuf.at[i], ssem).wait()
            gds = []
            for i in range(NB):
                s = k * NB + i
                gds.append(pltpu.async_copy(
                    ucur.at[colvb.at[s]], lbuf.at[i], gsem))
            for i in range(NB):
                s = k * NB + i
                gds[i].wait()

                @pl.loop(0, SUB, unroll=4)
                def _(j):
                    a = _splat2(attrvb, s, j)
                    lbuf[i, j, :] = lbuf[i, j, :] * a

                pltpu.async_copy(lbuf.at[i], acc16.at[rowvb.at[s]],
                                 ssem, add=True)

        for i in range(NB):
            pltpu.make_async_copy(
                u_hbm.at[pl.ds(0, SUB)], lbuf.at[i], ssem).wait()

    plsc.subcore_barrier()
    pltpu.sync_copy(acc16.at[pl.ds(wbase, ROWS_W)], wb)

    b1r = b1v[...]

    @pl.loop(0, ROWS_W, unroll=4)
    def _(r):
        di = plsc.load_gather(dinv, [jnp.full((L,), 0, _i32) + r])
        wb[r, :] = _sigmoid(wb[r, :] * di + b1r)

    pltpu.sync_copy(wb, out_hbm.at[pl.ds(wbase, ROWS_W)])

  return _sc_out


def kernel(x, soft_labels, edge_index, edge_attr, W0, b0, W1, b1):
    row2 = edge_index[0].reshape(EROWS, SUB)
    col2 = edge_index[1].reshape(EROWS, SUB)
    attr2 = edge_attr.reshape(EROWS, SUB)
    xp = jnp.pad(x, ((0, NPAD - N), (0, 0)))
    labp = jnp.pad(soft_labels, ((0, NPAD - N), (0, 0)))

    z0 = _tc_matmul(xp, W0)
    z0f = jnp.concatenate([z0[:, :D_Y], z0[:, D_Y:]], axis=0)
    y1p, labout, deginv = _build_sc_main()(row2, col2, attr2, z0f, labp)
    u = _tc_hidden(y1p[:NPAD], y1p[NPAD:], deginv.reshape(NPAD, 1),
                   b0.reshape(1, D_HID), W1)
    outp = _build_sc_out()(row2, col2, attr2, u, deginv, b1)
    return outp[:N], labout[:N]


# speedup vs baseline: 1.2537x; 1.0121x over previous
import functools

import jax
import jax.numpy as jnp
from jax import lax
from jax.experimental import pallas as pl
from jax.experimental.pallas import tpu as pltpu
from jax.experimental.pallas import tpu_sc as plsc

N = 10000
NPAD = 10240
E = 320000
D_IN = 128
D_HID = 128
D_OUT = 16
LPA_ITER = 3

NC = 2
NS = 16
L = 16
ROWS_T = NPAD // NS
ROWS_W = NPAD // (NC * NS)

SUB = 100
HROWS = 50
EROWS = E // SUB
NQ = 4
NB = 10
NBY = 5
D_Y = D_HID // NC

_i32 = jnp.int32
_f32 = jnp.float32


def _mm_body(x_ref, w_ref, o_ref):
    o_ref[...] = jnp.dot(x_ref[...], w_ref[0],
                         preferred_element_type=_f32)[None]


def _tc_matmul_split(x, ws, bn=1024):
    m, k = x.shape
    return pl.pallas_call(
        _mm_body,
        grid=(m // bn, NC),
        in_specs=[
            pl.BlockSpec((bn, k), lambda i, c: (i, 0)),
            pl.BlockSpec((1, k, D_Y), lambda i, c: (c, 0, 0)),
        ],
        out_specs=pl.BlockSpec((1, bn, D_Y), lambda i, c: (c, i, 0)),
        out_shape=jax.ShapeDtypeStruct((NC, m, D_Y), _f32),
    )(x, ws)


def _hid_body(pa_ref, pb_ref, di_ref, b0_ref, w1_ref, u_ref):
    h = jnp.concatenate([pa_ref[...], pb_ref[...]], axis=1)
    h = di_ref[...] * h + b0_ref[...]
    h = jnp.maximum(h, 0.0)
    u_ref[...] = jnp.dot(h, w1_ref[...], preferred_element_type=_f32)


def _tc_hidden(pa, pb, di, b0, w1, bn=1024):
    return pl.pallas_call(
        _hid_body,
        grid=(NPAD // bn,),
        in_specs=[
            pl.BlockSpec((bn, D_Y), lambda i: (i, 0)),
            pl.BlockSpec((bn, D_Y), lambda i: (i, 0)),
            pl.BlockSpec((bn, 1), lambda i: (i, 0)),
            pl.BlockSpec((1, D_HID), lambda i: (0, 0)),
            pl.BlockSpec((D_HID, D_OUT), lambda i: (0, 0)),
        ],
        out_specs=pl.BlockSpec((bn, D_OUT), lambda i: (i, 0)),
        out_shape=jax.ShapeDtypeStruct((NPAD, D_OUT), _f32),
    )(pa, pb, di, b0, w1)


def _splat2(ref2d, s, j):
    zi = jnp.full((L,), 0, _i32)
    return plsc.load_gather(ref2d, [zi + s, zi + j])


def _sigmoid(v):
    return 1.0 / (1.0 + jnp.exp(-v))


@functools.cache
def _mesh():
    return plsc.VectorSubcoreMesh(core_axis_name="c", subcore_axis_name="s",
                                  num_cores=NC, num_subcores=NS)


_sc_params = pltpu.CompilerParams(needs_layout_passes=False,
                                  use_tc_tiling_on_sc=False)


@functools.cache
def _build_sc_main():
  @functools.partial(
    pl.kernel,
    out_type=(
        jax.ShapeDtypeStruct((NC * NPAD, D_Y), _f32),
        jax.ShapeDtypeStruct((NPAD, D_OUT), _f32),
        jax.ShapeDtypeStruct((NPAD,), _f32),
    ),
    scratch_types=(
        pltpu.VMEM_SHARED((NPAD, D_Y), _f32),
        pltpu.VMEM_SHARED((NPAD, D_OUT), _f32),
        pltpu.VMEM_SHARED((NPAD, D_OUT), _f32),
        pltpu.VMEM_SHARED((NPAD,), _f32),
        pltpu.VMEM((HROWS, SUB), _i32),
        pltpu.VMEM((HROWS, SUB), _i32),
        pltpu.VMEM((HROWS, SUB), _f32),
        pltpu.VMEM((NB, SUB), _i32),
        pltpu.VMEM((NB, SUB, D_OUT), _f32),
        pltpu.VMEM((NBY, SUB, D_Y), _f32),
        pltpu.VMEM((ROWS_T,), _f32),
        pltpu.VMEM((ROWS_T // 4, D_OUT), _f32),
        pltpu.SemaphoreType.DMA,
        pltpu.SemaphoreType.DMA,
        pltpu.SemaphoreType.DMA,
        pltpu.SemaphoreType.DMA,
    ),
    mesh=_mesh(),
    compiler_params=_sc_params,
  )
  def _sc_main(row2_hbm, col2_hbm, attr2_hbm, z0f_hbm, lab0_hbm,
               y1p_hbm, labout_hbm, deginv_hbm,
               accy, acc16, labcur, degacc, rowvb, colvb, attrvb, colv2b,
               lbuf, ybuf, dinv, wb, esem, gsem, ssem, dsem):
    cid = lax.axis_index("c")
    sid = lax.axis_index("s")
    rbase = sid * ROWS_T
    z16 = jnp.zeros((L,), _f32)

    def _load_rows(base):
        d1 = pltpu.async_copy(row2_hbm.at[pl.ds(base, HROWS)], rowvb, esem)
        d2 = pltpu.async_copy(col2_hbm.at[pl.ds(base, HROWS)], colvb, esem)
        d3 = pltpu.async_copy(attr2_hbm.at[pl.ds(base, HROWS)], attrvb, esem)
        d1.wait()
        d2.wait()
        d3.wait()

    def _zero_acc16_slice():
        @pl.loop(0, SUB)
        def _(r):
            lbuf[0, r, :] = z16

        for off in (0, 100, 200, 300, 400, 500, 540):
            pltpu.sync_copy(lbuf.at[0], acc16.at[pl.ds(rbase + off, SUB)])

    def _lab_sweep(with_deg):
        @pl.loop(0, HROWS // NB)
        def _(k):
            @pl.when(k > 0)
            def _():
                for i in range(NB):
                    pltpu.make_async_copy(
                        lab0_hbm.at[pl.ds(0, SUB)], lbuf.at[i], ssem).wait()
            gds, dds = [], []
            for i in range(NB):
                s = k * NB + i
                gds.append(pltpu.async_copy(
                    labcur.at[colvb.at[s]], lbuf.at[i], gsem))
                if with_deg:
                    dds.append(pltpu.async_copy(
                        attrvb.at[s], degacc.at[rowvb.at[s]], dsem, add=True))
            for i in range(NB):
                s = k * NB + i
                gds[i].wait()

                @pl.loop(0, SUB, unroll=4)
                def _(j):
                    a = _splat2(attrvb, s, j)
                    lbuf[i, j, :] = lbuf[i, j, :] * a

                pltpu.async_copy(lbuf.at[i], acc16.at[rowvb.at[s]],
                                 ssem, add=True)
            for d in dds:
                d.wait()

    def _lab_drain():
        for i in range(NB):
            pltpu.make_async_copy(
                lab0_hbm.at[pl.ds(0, SUB)], lbuf.at[i], ssem).wait()

    @pl.loop(0, ROWS_T // L)
    def _(m):
        dinv[pl.ds(m * L, L)] = z16

    @pl.loop(0, SUB)
    def _(r):
        for d in range(D_Y // L):
            ybuf[0, r, pl.ds(d * L, L)] = z16

    pltpu.sync_copy(dinv, degacc.at[pl.ds(rbase, ROWS_T)])
    _zero_acc16_slice()
    for off in (0, 100, 200, 300, 400, 500, 540):
        pltpu.sync_copy(ybuf.at[0], accy.at[pl.ds(rbase + off, SUB)])

    for p in range(4):
        pb = p * (ROWS_T // 4)
        pltpu.sync_copy(lab0_hbm.at[pl.ds(rbase + pb, ROWS_T // 4)], wb)
        pltpu.sync_copy(wb, labcur.at[pl.ds(rbase + pb, ROWS_T // 4)])
    plsc.subcore_barrier()

    @pl.loop(0, NQ)
    def _(h):
        _load_rows(sid * NQ * HROWS + h * HROWS)
        _lab_sweep(True)
        _lab_drain()

    plsc.subcore_barrier()

    pltpu.sync_copy(degacc.at[pl.ds(rbase, ROWS_T)], dinv)

    @pl.loop(0, ROWS_T // L)
    def _(m):
        s = pl.ds(m * L, L)
        d = dinv[s]
        dinv[s] = jnp.where(d > 0.0, 1.0 / d, 0.0)

    @pl.when(cid == 0)
    def _():
        pltpu.sync_copy(dinv, deginv_hbm.at[pl.ds(rbase, ROWS_T)])

    for it in range(LPA_ITER):
        if it > 0:
            @pl.loop(0, NQ)
            def _(h):
                _load_rows(sid * NQ * HROWS + h * HROWS)
                _lab_sweep(False)
                _lab_drain()

            plsc.subcore_barrier()

        for p in range(4):
            pb = p * (ROWS_T // 4)
            pltpu.sync_copy(acc16.at[pl.ds(rbase + pb, ROWS_T // 4)], wb)
            if it == LPA_ITER - 1:
                @pl.loop(0, ROWS_T // 4, unroll=4)
                def _(r):
                    di = plsc.load_gather(
                        dinv, [jnp.full((L,), pb, _i32) + r])
                    wb[r, :] = _sigmoid(wb[r, :] * di)

                @pl.when(cid == 0)
                def _():
                    pltpu.sync_copy(
                        wb, labout_hbm.at[pl.ds(rbase + pb, ROWS_T // 4)])
            else:
                @pl.loop(0, ROWS_T // 4, unroll=4)
                def _(r):
                    di = plsc.load_gather(
                        dinv, [jnp.full((L,), pb, _i32) + r])
                    wb[r, :] = wb[r, :] * di

                pltpu.sync_copy(
                    wb, labcur.at[pl.ds(rbase + pb, ROWS_T // 4)])
        if it < LPA_ITER - 1:
            _zero_acc16_slice()
        plsc.subcore_barrier()

    @pl.loop(0, NQ)
    def _(h):
        _load_rows(sid * NQ * HROWS + h * HROWS)

        @pl.loop(0, HROWS // NBY)
        def _(k):
            @pl.when(k > 0)
            def _():
                for i in range(NBY):
                    pltpu.make_async_copy(
                        z0f_hbm.at[pl.ds(0, SUB)], ybuf.at[i], ssem).wait()
            gds = []
            for i in range(NBY):
                s = k * NBY + i

                @pl.loop(0, 7)
                def _(m):
                    sl = pl.ds(jnp.minimum(m * L, SUB - L), L)
                    colv2b[i, sl] = colvb[s, sl] + cid * NPAD

                gds.append(pltpu.async_copy(
                    z0f_hbm.at[colv2b.at[i]], ybuf.at[i], gsem))
            for i in range(NBY):
                s = k * NBY + i
                gds[i].wait()

                @pl.loop(0, SUB, unroll=4)
                def _(j):
                    a = _splat2(attrvb, s, j)
                    for d in range(D_Y // L):
                        sl = pl.ds(d * L, L)
                        ybuf[i, j, sl] = ybuf[i, j, sl] * a

                pltpu.async_copy(ybuf.at[i], accy.at[rowvb.at[s]],
                                 ssem, add=True)

        for i in range(NBY):
            pltpu.make_async_copy(
                z0f_hbm.at[pl.ds(0, SUB)], ybuf.at[i], ssem).wait()

    plsc.subcore_barrier()
    pltpu.sync_copy(accy.at[pl.ds(rbase, ROWS_T)],
                    y1p_hbm.at[pl.ds(cid * NPAD + rbase, ROWS_T)])

  return _sc_main


@functools.cache
def _build_sc_out():
  @functools.partial(
    pl.kernel,
    out_type=jax.ShapeDtypeStruct((NPAD, D_OUT), _f32),
    scratch_types=(
        pltpu.VMEM_SHARED((NPAD, D_OUT), _f32),
        pltpu.VMEM_SHARED((NPAD, D_OUT), _f32),
        pltpu.VMEM((HROWS, SUB), _i32),
        pltpu.VMEM((HROWS, SUB), _i32),
        pltpu.VMEM((HROWS, SUB), _f32),
        pltpu.VMEM((NB, SUB, D_OUT), _f32),
        pltpu.VMEM((160, D_OUT), _f32),
        pltpu.VMEM((ROWS_W,), _f32),
        pltpu.VMEM((ROWS_W, D_OUT), _f32),
        pltpu.VMEM((L,), _f32),
        pltpu.SemaphoreType.DMA,
        pltpu.SemaphoreType.DMA,
        pltpu.SemaphoreType.DMA,
    ),
    mesh=_mesh(),
    compiler_params=_sc_params,
  )
  def _sc_out(row2_hbm, col2_hbm, attr2_hbm, u_hbm, deginv_hbm,
              b1_hbm, out_hbm, acc16, ucur, rowvb, colvb, attrvb, lbuf,
              zlab, dinv, wb, b1v, esem, gsem, ssem):
    cid = lax.axis_index("c")
    sid = lax.axis_index("s")
    rbase = sid * ROWS_T
    wbase = cid * (NPAD // NC) + sid * ROWS_W
    z16 = jnp.zeros((L,), _f32)

    @pl.loop(0, 160)
    def _(r):
        zlab[r, :] = z16

    for i in range(4):
        pltpu.sync_copy(zlab, acc16.at[pl.ds(rbase + i * 160, 160)])
    for p in range(2):
        pb = p * ROWS_W
        pltpu.sync_copy(u_hbm.at[pl.ds(rbase + pb, ROWS_W)], wb)
        pltpu.sync_copy(wb, ucur.at[pl.ds(rbase + pb, ROWS_W)])
    pltpu.sync_copy(deginv_hbm.at[pl.ds(wbase, ROWS_W)], dinv)
    pltpu.sync_copy(b1_hbm, b1v)
    plsc.subcore_barrier()

    for h in range(NQ):
        d1 = pltpu.async_copy(
            row2_hbm.at[pl.ds(sid * NQ * HROWS + h * HROWS, HROWS)], rowvb, esem)
        d2 = pltpu.async_copy(
            col2_hbm.at[pl.ds(sid * NQ * HROWS + h * HROWS, HROWS)], colvb, esem)
        d3 = pltpu.async_copy(
            attr2_hbm.at[pl.ds(sid * NQ * HROWS + h * HROWS, HROWS)], attrvb, esem)
        d1.wait()
        d2.wait()
        d3.wait()

        @pl.loop(0, HROWS // NB)
        def _(k):
            @pl.when(k > 0)
            def _():
                for i in range(NB):
                    pltpu.make_async_copy(
                        u_hbm.at[pl.ds(0, SUB)], lbuf.at[i], ssem).wait()
            gds = []
            for i in range(NB):
                s = k * NB + i
                gds.append(pltpu.async_copy(
                    ucur.at[colvb.at[s]], lbuf.at[i], gsem))
            for i in range(NB):
                s = k * NB + i
                gds[i].wait()

                @pl.loop(0, SUB, unroll=4)
                def _(j):
                    a = _splat2(attrvb, s, j)
                    lbuf[i, j, :] = lbuf[i, j, :] * a

                pltpu.async_copy(lbuf.at[i], acc16.at[rowvb.at[s]],
                                 ssem, add=True)

        for i in range(NB):
            pltpu.make_async_copy(
                u_hbm.at[pl.ds(0, SUB)], lbuf.at[i], ssem).wait()

    plsc.subcore_barrier()
    pltpu.sync_copy(acc16.at[pl.ds(wbase, ROWS_W)], wb)

    b1r = b1v[...]

    @pl.loop(0, ROWS_W, unroll=4)
    def _(r):
        di = plsc.load_gather(dinv, [jnp.full((L,), 0, _i32) + r])
        wb[r, :] = _sigmoid(wb[r, :] * di + b1r)

    pltpu.sync_copy(wb, out_hbm.at[pl.ds(wbase, ROWS_W)])

  return _sc_out


def kernel(x, soft_labels, edge_index, edge_attr, W0, b0, W1, b1):
    row2 = edge_index[0].reshape(EROWS, SUB)
    col2 = edge_index[1].reshape(EROWS, SUB)
    attr2 = edge_attr.reshape(EROWS, SUB)
    xp = jnp.pad(x, ((0, NPAD - N), (0, 0)))
    labp = jnp.pad(soft_labels, ((0, NPAD - N), (0, 0)))

    w0s = W0.reshape(D_IN, NC, D_Y).transpose(1, 0, 2)
    z0f = _tc_matmul_split(xp, w0s).reshape(NC * NPAD, D_Y)
    y1p, labout, deginv = _build_sc_main()(row2, col2, attr2, z0f, labp)
    u = _tc_hidden(y1p[:NPAD], y1p[NPAD:], deginv.reshape(NPAD, 1),
                   b0.reshape(1, D_HID), W1)
    outp = _build_sc_out()(row2, col2, attr2, u, deginv, b1)
    return outp[:N], labout[:N]
